# baseline (speedup 1.0000x reference)
import functools

import jax
import jax.numpy as jnp
from jax import lax
from jax.experimental import pallas as pl
from jax.experimental.pallas import tpu as pltpu
from jax.experimental.pallas import tpu_sc as plsc

B, D, C = 4096, 2048, 512
TAU = 0.5
EPS = 1e-12

NC, NS, L = 2, 16, 16
NW = NC * NS
ROWS_PER_TILE = B // NW
CHUNK = 32
NCHUNK = ROWS_PER_TILE // CHUNK
DH = D // 2
UNROLL = 8


def _rsqrt_vec(s):
    i = plsc.bitcast(s, jnp.int32)
    i = jnp.int32(0x5F3759DF) - lax.shift_right_logical(i, 1)
    y = plsc.bitcast(i, jnp.float32)
    for _ in range(3):
        y = y * (1.5 - 0.5 * s * y * y)
    return y


def _sc_body(fvp, fap, frp, label, out, acc, rowsh, rowsb, zbuf, ybuf, idx):
    cid = lax.axis_index("c")
    sid = lax.axis_index("s")
    zeros16 = jnp.zeros((L,), jnp.float32)
    lane = lax.broadcasted_iota(jnp.int32, (L,), 0)

    def zb_body(j, _):
        base = pl.multiple_of(j * L, L)
        for r in range(8):
            zbuf[r, pl.ds(base, L)] = zeros16
        return 0
    lax.fori_loop(0, DH // L, zb_body, 0)

    my_rows0 = (cid * NS + sid) * ROWS_PER_TILE
    acc_r0 = sid * (C // NS)

    for f, feat in enumerate((fvp, fap, frp)):
        for h in range(2):
            for z in range(4):
                pltpu.sync_copy(zbuf, acc.at[pl.ds(acc_r0 + 8 * z, 8)])
            plsc.subcore_barrier()

            def chunk_body(g, _):
                row0 = my_rows0 + g * CHUNK
                pltpu.sync_copy(label.at[pl.ds(row0, CHUNK)], idx)
                pltpu.sync_copy(
                    feat.at[pl.ds(row0, CHUNK), pl.ds(h * DH, DH)], rowsh)
                if h == 0:
                    pltpu.sync_copy(
                        feat.at[pl.ds(row0, CHUNK), pl.ds(DH, DH)], rowsb)

                for rg in range(CHUNK // L):
                    glane = lane + rg * L
                    ybase = pl.multiple_of(g * CHUNK + rg * L, L)
                    if h == 0:
                        def nrm_body(j, carry):
                            a, col = carry
                            for _ in range(UNROLL):
                                vl = plsc.load_gather(rowsh, [glane, col])
                                vr = plsc.load_gather(rowsb, [glane, col])
                                a = a + vl * vl + vr * vr
                                col = col + 1
                            return a, col
                        a, _ = lax.fori_loop(0, DH // UNROLL, nrm_body,
                                             (jnp.zeros((L,), jnp.float32),
                                              jnp.zeros((L,), jnp.int32)))
                        y = _rsqrt_vec(a)
                        ybuf[0, pl.ds(ybase, L)] = y
                    else:
                        y = ybuf[0, pl.ds(ybase, L)]

                    def scl_body(j, col):
                        for _ in range(UNROLL):
                            v = plsc.load_gather(rowsh, [glane, col])
                            plsc.store_scatter(rowsh, [glane, col], v * y)
                            col = col + 1
                        return col
                    lax.fori_loop(0, DH // UNROLL, scl_body,
                                  jnp.zeros((L,), jnp.int32))

                pltpu.sync_copy(rowsh, acc.at[idx], add=True)
                return 0
            lax.fori_loop(0, NCHUNK, chunk_body, 0)

            plsc.subcore_barrier()
            pltpu.sync_copy(
                acc.at[pl.ds(acc_r0, C // NS)],
                out.at[cid, f, pl.ds(acc_r0, C // NS), pl.ds(h * DH, DH)])


@functools.partial(jax.jit, static_argnames=())
def _sc_segment_sums(feat_vp, feat_ap, feat_rp, label):
    mesh = plsc.VectorSubcoreMesh(core_axis_name="c", subcore_axis_name="s",
                                  num_cores=NC, num_subcores=NS)
    return pl.kernel(
        _sc_body,
        out_type=jax.ShapeDtypeStruct((NC, 3, C, D), jnp.float32),
        mesh=mesh,
        compiler_params=pltpu.CompilerParams(needs_layout_passes=False,
                                             use_tc_tiling_on_sc=False),
        scratch_types=[
            pltpu.VMEM_SHARED((C, DH), jnp.float32),
            pltpu.VMEM((CHUNK, DH), jnp.float32),
            pltpu.VMEM((CHUNK, DH), jnp.float32),
            pltpu.VMEM((8, DH), jnp.float32),
            pltpu.VMEM((1, ROWS_PER_TILE), jnp.float32),
            pltpu.VMEM((CHUNK,), jnp.int32),
        ],
    )(feat_vp, feat_ap, feat_rp, label)


def _tc_body(partials_ref, label_ref, out_ref):
    lab = label_ref[...]
    oh = (lab == lax.broadcasted_iota(jnp.int32, (B, C), 1))
    counts = jnp.sum(oh.astype(jnp.float32), axis=0)
    n = jnp.maximum(counts, 1.0)
    present = counts > 0.0
    n_present = jnp.sum(jnp.where(present, 1.0, 0.0))

    total = jnp.float32(0.0)
    us = []
    for f in range(3):
        s = partials_ref[0, f] + partials_ref[1, f]
        ns2 = jnp.sum(s * s, axis=1)
        inv = 1.0 / jnp.maximum(jnp.sqrt(ns2), n * EPS)
        total += (B + jnp.sum(counts * ns2 * inv * inv)
                  - 2.0 * jnp.sum(ns2 * inv)) / B
        us.append(s * inv[:, None])

    rows_i = lax.broadcasted_iota(jnp.int32, (C, C), 0)
    cols_i = lax.broadcasted_iota(jnp.int32, (C, C), 1)
    eye = rows_i == cols_i
    for a, b in ((0, 1), (0, 2), (1, 2)):
        g = lax.dot_general(us[a], us[b], (((1,), (1,)), ((), ())),
                            preferred_element_type=jnp.float32) / TAU
        diag = jnp.sum(jnp.where(eye, g, 0.0), axis=1)
        gm = jnp.where(present[None, :], g, -jnp.inf)
        m = jnp.max(gm, axis=1)
        lse = m + jnp.log(jnp.sum(jnp.exp(gm - m[:, None]), axis=1))
        total += jnp.sum(jnp.where(present, lse - diag, 0.0)) / n_present

    out_ref[0, 0] = total


def _tc_finish(partials, lab2d):
    return pl.pallas_call(
        _tc_body,
        out_shape=jax.ShapeDtypeStruct((1, 1), jnp.float32),
        out_specs=pl.BlockSpec(memory_space=pltpu.SMEM),
    )(partials, lab2d)


def kernel(feat_vp, feat_ap, feat_rp, label):
    partials = _sc_segment_sums(feat_vp, feat_ap, feat_rp, label)
    loss = _tc_finish(partials, label.reshape(B, 1))
    return loss[0, 0]


# speedup vs baseline: 5.4345x; 5.4345x over previous
import functools

import jax
import jax.numpy as jnp
from jax import lax
from jax.experimental import pallas as pl
from jax.experimental.pallas import tpu as pltpu
from jax.experimental.pallas import tpu_sc as plsc

B, D, C = 4096, 2048, 512
TAU = 0.5
EPS = 1e-12

NC, NS, L = 2, 16, 16
NW = NC * NS
ROWS_PER_TILE = B // NW
CHUNK = 32
NCHUNK = ROWS_PER_TILE // CHUNK
DH = D // 2
UNROLL = 8


def _rsqrt_vec(s):
    i = plsc.bitcast(s, jnp.int32)
    i = jnp.int32(0x5F3759DF) - lax.shift_right_logical(i, 1)
    y = plsc.bitcast(i, jnp.float32)
    for _ in range(3):
        y = y * (1.5 - 0.5 * s * y * y)
    return y


def _sc_body(fvp, fap, frp, label, out, acc, buf0, buf1, zbuf, idx,
             sem0, sem1):
    cid = lax.axis_index("c")
    sid = lax.axis_index("s")
    zeros16 = jnp.zeros((L,), jnp.float32)

    def zb_body(j, _):
        base = pl.multiple_of(j * L, L)
        for r in range(8):
            zbuf[r, pl.ds(base, L)] = zeros16
        return 0
    lax.fori_loop(0, DH // L, zb_body, 0)

    my_rows0 = (cid * NS + sid) * ROWS_PER_TILE
    acc_r0 = sid * (C // NS)

    for g in range(NCHUNK):
        pltpu.sync_copy(label.at[pl.ds(my_rows0 + g * CHUNK, CHUNK)],
                        idx.at[g])

    bufs = (buf0, buf1)
    sems = (sem0, sem1)
    for f, feat in enumerate((fvp, fap, frp)):
        for h in range(2):
            for z in range(4):
                pltpu.sync_copy(zbuf, acc.at[pl.ds(acc_r0 + 8 * z, 8)])
            plsc.subcore_barrier()

            def load(g):
                row0 = my_rows0 + g * CHUNK
                return pltpu.async_copy(
                    feat.at[pl.ds(row0, CHUNK), pl.ds(h * DH, DH)],
                    bufs[g % 2], sems[g % 2])

            pend = [load(0), load(1)]
            for g in range(NCHUNK):
                pend[g % 2].wait()
                pltpu.sync_copy(bufs[g % 2], acc.at[idx.at[g]], add=True)
                if g + 2 < NCHUNK:
                    pend[g % 2] = load(g + 2)

            plsc.subcore_barrier()
            pltpu.sync_copy(
                acc.at[pl.ds(acc_r0, C // NS)],
                out.at[cid, f, pl.ds(acc_r0, C // NS), pl.ds(h * DH, DH)])


@functools.partial(jax.jit, static_argnames=())
def _sc_segment_sums(feat_vp, feat_ap, feat_rp, label):
    mesh = plsc.VectorSubcoreMesh(core_axis_name="c", subcore_axis_name="s",
                                  num_cores=NC, num_subcores=NS)
    return pl.kernel(
        _sc_body,
        out_type=jax.ShapeDtypeStruct((NC, 3, C, D), jnp.float32),
        mesh=mesh,
        compiler_params=pltpu.CompilerParams(needs_layout_passes=False,
                                             use_tc_tiling_on_sc=False),
        scratch_types=[
            pltpu.VMEM_SHARED((C, DH), jnp.float32),
            pltpu.VMEM((CHUNK, DH), jnp.float32),
            pltpu.VMEM((CHUNK, DH), jnp.float32),
            pltpu.VMEM((8, DH), jnp.float32),
            pltpu.VMEM((NCHUNK, CHUNK), jnp.int32),
            pltpu.SemaphoreType.DMA,
            pltpu.SemaphoreType.DMA,
        ],
    )(feat_vp, feat_ap, feat_rp, label)


def _tc_norm_body(x_ref, o_ref):
    x = x_ref[...]
    s = jnp.sum(x * x, axis=1, keepdims=True)
    o_ref[...] = x * lax.rsqrt(jnp.maximum(s, 1e-24))


def _tc_normalize(x):
    blk = 512
    return pl.pallas_call(
        _tc_norm_body,
        grid=(B // blk,),
        in_specs=[pl.BlockSpec((blk, D), lambda i: (i, 0))],
        out_specs=pl.BlockSpec((blk, D), lambda i: (i, 0)),
        out_shape=jax.ShapeDtypeStruct((B, D), jnp.float32),
    )(x)


def _tc_body(partials_ref, label_ref, out_ref):
    lab = label_ref[...]
    oh = (lab == lax.broadcasted_iota(jnp.int32, (B, C), 1))
    counts = jnp.sum(oh.astype(jnp.float32), axis=0)
    n = jnp.maximum(counts, 1.0)
    present = counts > 0.0
    n_present = jnp.sum(jnp.where(present, 1.0, 0.0))

    total = jnp.float32(0.0)
    us = []
    for f in range(3):
        s = partials_ref[0, f] + partials_ref[1, f]
        ns2 = jnp.sum(s * s, axis=1)
        inv = 1.0 / jnp.maximum(jnp.sqrt(ns2), n * EPS)
        total += (B + jnp.sum(counts * ns2 * inv * inv)
                  - 2.0 * jnp.sum(ns2 * inv)) / B
        us.append(s * inv[:, None])

    rows_i = lax.broadcasted_iota(jnp.int32, (C, C), 0)
    cols_i = lax.broadcasted_iota(jnp.int32, (C, C), 1)
    eye = rows_i == cols_i
    for a, b in ((0, 1), (0, 2), (1, 2)):
        g = lax.dot_general(us[a], us[b], (((1,), (1,)), ((), ())),
                            preferred_element_type=jnp.float32) / TAU
        diag = jnp.sum(jnp.where(eye, g, 0.0), axis=1)
        gm = jnp.where(present[None, :], g, -jnp.inf)
        m = jnp.max(gm, axis=1)
        lse = m + jnp.log(jnp.sum(jnp.exp(gm - m[:, None]), axis=1))
        total += jnp.sum(jnp.where(present, lse - diag, 0.0)) / n_present

    out_ref[0, 0] = total


def _tc_finish(partials, lab2d):
    return pl.pallas_call(
        _tc_body,
        out_shape=jax.ShapeDtypeStruct((1, 1), jnp.float32),
        out_specs=pl.BlockSpec(memory_space=pltpu.SMEM),
    )(partials, lab2d)


def kernel(feat_vp, feat_ap, feat_rp, label):
    nvp = _tc_normalize(feat_vp)
    nap = _tc_normalize(feat_ap)
    nrp = _tc_normalize(feat_rp)
    partials = _sc_segment_sums(nvp, nap, nrp, label)
    loss = _tc_finish(partials, label.reshape(B, 1))
    return loss[0, 0]


# speedup vs baseline: 8.4791x; 1.5602x over previous
import functools

import jax
import jax.numpy as jnp
from jax import lax
from jax.experimental import pallas as pl
from jax.experimental.pallas import tpu as pltpu
from jax.experimental.pallas import tpu_sc as plsc

B, D, C = 4096, 2048, 512
TAU = 0.5
EPS = 1e-12

NC, NS, L = 2, 16, 16
NW = NC * NS
ROWS_PER_TILE = B // NW
CHUNK = 32
NCHUNK = ROWS_PER_TILE // CHUNK
DH = D // 2


def _sc_body(feat, label, out, acc, buf0, buf1, zbuf, idx,
             sem0, sem1, ssem0, ssem1):
    cid = lax.axis_index("c")
    sid = lax.axis_index("s")
    zeros16 = jnp.zeros((L,), jnp.float32)

    def zb_body(j, _):
        base = pl.multiple_of(j * L, L)
        for r in range(8):
            for jj in range(DH // 128):
                zbuf[r, jj, pl.ds(base, L)] = zeros16
        return 0
    lax.fori_loop(0, 128 // L, zb_body, 0)

    my_rows0 = (cid * NS + sid) * ROWS_PER_TILE
    acc_r0 = sid * (C // NS)

    for g in range(NCHUNK):
        pltpu.sync_copy(label.at[pl.ds(my_rows0 + g * CHUNK, CHUNK)],
                        idx.at[g])

    bufs = (buf0, buf1)
    sems = (sem0, sem1)
    ssems = (ssem0, ssem1)
    for h in range(2):
        for z in range(4):
            pltpu.sync_copy(zbuf, acc.at[pl.ds(acc_r0 + 8 * z, 8)])
        plsc.subcore_barrier()

        def load(g):
            row0 = my_rows0 + g * CHUNK
            return pltpu.async_copy(
                feat.at[pl.ds(row0, CHUNK), pl.ds(h * (DH // 128), DH // 128)],
                bufs[g % 2], sems[g % 2])

        pend = [load(0), load(1)]
        scat = [None, None]
        for g in range(NCHUNK):
            pend[g % 2].wait()
            scat[g % 2] = pltpu.async_copy(
                bufs[g % 2], acc.at[idx.at[g]], ssems[g % 2], add=True)
            if g + 2 < NCHUNK:
                scat[g % 2].wait()
                pend[g % 2] = load(g + 2)
        for g in (NCHUNK - 2, NCHUNK - 1):
            scat[g % 2].wait()

        plsc.subcore_barrier()
        pltpu.sync_copy(
            acc.at[pl.ds(acc_r0, C // NS)],
            out.at[cid, pl.ds(acc_r0, C // NS),
                   pl.ds(h * (DH // 128), DH // 128)])


@functools.partial(jax.jit, static_argnames=())
def _sc_segment_sums(feat, label):
    mesh = plsc.VectorSubcoreMesh(core_axis_name="c", subcore_axis_name="s",
                                  num_cores=NC, num_subcores=NS)
    return pl.kernel(
        _sc_body,
        out_type=jax.ShapeDtypeStruct((NC, C, D // 128, 128), jnp.float32),
        mesh=mesh,
        compiler_params=pltpu.CompilerParams(needs_layout_passes=False,
                                             use_tc_tiling_on_sc=False),
        scratch_types=[
            pltpu.VMEM_SHARED((C, DH // 128, 128), jnp.float32),
            pltpu.VMEM((CHUNK, DH // 128, 128), jnp.float32),
            pltpu.VMEM((CHUNK, DH // 128, 128), jnp.float32),
            pltpu.VMEM((8, DH // 128, 128), jnp.float32),
            pltpu.VMEM((NCHUNK, CHUNK), jnp.int32),
            pltpu.SemaphoreType.DMA,
            pltpu.SemaphoreType.DMA,
            pltpu.SemaphoreType.DMA,
            pltpu.SemaphoreType.DMA,
        ],
    )(feat, label)


def _tc_norm_body(x_ref, o_ref):
    x = x_ref[...]
    s = jnp.sum(x * x, axis=1, keepdims=True)
    xn = x * lax.rsqrt(jnp.maximum(s, 1e-24))
    o_ref[...] = xn.reshape(o_ref.shape)


def _tc_normalize(x):
    blk = 512
    return pl.pallas_call(
        _tc_norm_body,
        grid=(B // blk,),
        in_specs=[pl.BlockSpec((blk, D), lambda i: (i, 0))],
        out_specs=pl.BlockSpec((blk, D // 128, 128), lambda i: (i, 0, 0)),
        out_shape=jax.ShapeDtypeStruct((B, D // 128, 128), jnp.float32),
    )(x)


def _counts_from(label_ref):
    lab = label_ref[...]
    oh = (lab == lax.broadcasted_iota(jnp.int32, (B, C), 1))
    return jnp.sum(oh.astype(jnp.float32), axis=0)


def _center_matrix(p_ref, counts):
    n = jnp.maximum(counts, 1.0)
    s = (p_ref[0] + p_ref[1]).reshape(C, D)
    ns2 = jnp.sum(s * s, axis=1)
    inv = 1.0 / jnp.maximum(jnp.sqrt(ns2), n * EPS)
    t = (B + jnp.sum(counts * ns2 * inv * inv)
         - 2.0 * jnp.sum(ns2 * inv)) / B
    return (s * inv[:, None]).astype(jnp.bfloat16), t


def _tc_prep_body(p_ref, label_ref, u_ref, t_ref):
    counts = _counts_from(label_ref)
    u, t = _center_matrix(p_ref, counts)
    t_ref[0, 0] = t
    u_ref[...] = u


def _tc_prep(partial, lab2d):
    return pl.pallas_call(
        _tc_prep_body,
        out_shape=(jax.ShapeDtypeStruct((C, D), jnp.bfloat16),
                   jax.ShapeDtypeStruct((1, 1), jnp.float32)),
        out_specs=(pl.BlockSpec(),
                   pl.BlockSpec(memory_space=pltpu.SMEM)),
    )(partial, lab2d)


def _tc_final_body(prp_ref, uvp_ref, uap_ref, tvp_ref, tap_ref,
                   label_ref, out_ref):
    counts = _counts_from(label_ref)
    present = counts > 0.0
    n_present = jnp.sum(jnp.where(present, 1.0, 0.0))
    urp, trp = _center_matrix(prp_ref, counts)
    total = tvp_ref[0, 0] + tap_ref[0, 0] + trp

    us = (uvp_ref[...], uap_ref[...], urp)
    rows_i = lax.broadcasted_iota(jnp.int32, (C, C), 0)
    cols_i = lax.broadcasted_iota(jnp.int32, (C, C), 1)
    eye = rows_i == cols_i
    for a, b in ((0, 1), (0, 2), (1, 2)):
        g = lax.dot_general(us[a], us[b], (((1,), (1,)), ((), ())),
                            preferred_element_type=jnp.float32) / TAU
        diag = jnp.sum(jnp.where(eye, g, 0.0), axis=1)
        gm = jnp.where(present[None, :], g, -jnp.inf)
        m = jnp.max(gm, axis=1)
        lse = m + jnp.log(jnp.sum(jnp.exp(gm - m[:, None]), axis=1))
        total += jnp.sum(jnp.where(present, lse - diag, 0.0)) / n_present

    out_ref[0, 0] = total


def _tc_finish(prp, preps, lab2d):
    (uvp, tvp), (uap, tap) = preps
    return pl.pallas_call(
        _tc_final_body,
        out_shape=jax.ShapeDtypeStruct((1, 1), jnp.float32),
        in_specs=[pl.BlockSpec(), pl.BlockSpec(), pl.BlockSpec(),
                  pl.BlockSpec(memory_space=pltpu.SMEM),
                  pl.BlockSpec(memory_space=pltpu.SMEM),
                  pl.BlockSpec()],
        out_specs=pl.BlockSpec(memory_space=pltpu.SMEM),
    )(prp, uvp, uap, tvp, tap, lab2d)


def kernel(feat_vp, feat_ap, feat_rp, label):
    lab2d = label.reshape(B, 1)
    preps = [
        _tc_prep(_sc_segment_sums(_tc_normalize(f), label), lab2d)
        for f in (feat_vp, feat_ap)
    ]
    prp = _sc_segment_sums(_tc_normalize(feat_rp), label)
    loss = _tc_finish(prp, preps, lab2d)
    return loss[0, 0]
